# baseline (speedup 1.0000x reference)
import functools

import jax
import jax.numpy as jnp
from jax import lax
from jax.experimental import pallas as pl
from jax.experimental.pallas import tpu as pltpu
from jax.experimental.pallas import tpu_sc as plsc

HIDDEN = 768
INTER = 1024
NUM_EXPERTS = 64
TOPK = 2
S = 2048
SLOTS = S * TOPK
BT = 64
NT = NUM_EXPERTS + SLOTS // BT
NPAD = NT * BT
NW = 32
TPW = S // NW


def _sigmoid(x):
    return 1.0 / (1.0 + jnp.exp(-x))


def _router_body(hs_ref, rw_ref, scores_ref, pos_ref, meta_ref):
    x = hs_ref[...]
    w = rw_ref[...]
    logits = lax.dot_general(x, w, (((1,), (1,)), ((), ())),
                             preferred_element_type=jnp.float32)
    iota_e = lax.broadcasted_iota(jnp.int32, (S, NUM_EXPERTS), 1)
    big = jnp.int32(2**30)
    m0 = jnp.max(logits, axis=1)
    a0 = jnp.min(jnp.where(logits == m0[:, None], iota_e, big), axis=1)
    masked = jnp.where(iota_e == a0[:, None], -jnp.inf, logits)
    m1 = jnp.max(masked, axis=1)
    a1 = jnp.min(jnp.where(masked == m1[:, None], iota_e, big), axis=1)
    e1 = jnp.exp(m1 - m0)
    scores_ref[0, :] = 1.0 / (1.0 + e1)
    scores_ref[1, :] = e1 / (1.0 + e1)

    oh0 = (a0[:, None] == iota_e).astype(jnp.float32)
    oh1 = (a1[:, None] == iota_e).astype(jnp.float32)

    CB = 256
    r_i = lax.broadcasted_iota(jnp.int32, (CB, CB), 0)
    c_i = lax.broadcasted_iota(jnp.int32, (CB, CB), 1)
    l_incl = (r_i >= c_i).astype(jnp.float32)
    carry = jnp.zeros((1, NUM_EXPERTS), jnp.float32)
    ranks = []
    for half in (oh0, oh1):
        for k in range(S // CB):
            blk = half[k * CB:(k + 1) * CB, :]
            cum_in = lax.dot_general(l_incl, blk, (((1,), (0,)), ((), ())),
                                     preferred_element_type=jnp.float32) + carry
            rank_rows = cum_in - blk
            ranks.append(jnp.sum(rank_rows * blk, axis=1))
            carry = carry + jnp.sum(blk, axis=0, keepdims=True)
    counts = carry
    nchunk = S // CB

    n_e = jnp.floor((counts + (BT - 1)) * (1.0 / BT))
    iu_r = lax.broadcasted_iota(jnp.int32, (NUM_EXPERTS, NUM_EXPERTS), 0)
    iu_c = lax.broadcasted_iota(jnp.int32, (NUM_EXPERTS, NUM_EXPERTS), 1)
    u_incl = (iu_r <= iu_c).astype(jnp.float32)
    cumn_incl = lax.dot_general(n_e, u_incl, (((1,), (0,)), ((), ())),
                                preferred_element_type=jnp.float32)
    cumn_excl = cumn_incl - n_e
    base_e = cumn_excl * float(BT)

    base0 = jnp.sum(oh0 * base_e, axis=1)
    base1 = jnp.sum(oh1 * base_e, axis=1)
    for k in range(nchunk):
        sl = slice(k * CB, (k + 1) * CB)
        pos_ref[0, sl] = (base0[sl] + ranks[k]).astype(jnp.int32)
        pos_ref[1, sl] = (base1[sl] + ranks[nchunk + k]).astype(jnp.int32)

    jt = lax.broadcasted_iota(jnp.int32, (NT, NUM_EXPERTS), 0).astype(jnp.float32)
    owner = jnp.sum((jnp.broadcast_to(cumn_incl, (NT, NUM_EXPERTS)) <= jt)
                    .astype(jnp.int32), axis=1)
    valid = (owner < NUM_EXPERTS).astype(jnp.int32)
    iota_e1 = lax.broadcasted_iota(jnp.int32, (1, NUM_EXPERTS), 1)
    last_e = jnp.max(jnp.where(counts > 0.0, iota_e1, 0))
    meta_ref[0, :] = jnp.where(valid == 1, owner, last_e)
    meta_ref[1, :] = valid


def _router(hs2, router_w):
    return pl.pallas_call(
        _router_body,
        out_shape=(
            jax.ShapeDtypeStruct((2, S), jnp.float32),
            jax.ShapeDtypeStruct((2, S), jnp.int32),
            jax.ShapeDtypeStruct((2, NT), jnp.int32),
        ),
    )(hs2, router_w)


def _sc_dispatch(hs2, pos0, pos1):
    mesh = plsc.VectorSubcoreMesh(core_axis_name="c", subcore_axis_name="s")

    @functools.partial(
        pl.kernel, mesh=mesh,
        out_type=jax.ShapeDtypeStruct((NPAD, HIDDEN), jnp.float32),
        scratch_types=[
            pltpu.VMEM((TPW,), jnp.int32),
            pltpu.VMEM((TPW, HIDDEN), jnp.float32),
            pltpu.SemaphoreType.DMA,
        ],
    )
    def k(hs_hbm, p0_hbm, p1_hbm, xpad_hbm, idx_v, rows_v, sem):
        wid = lax.axis_index("s") * 2 + lax.axis_index("c")
        base = wid * TPW
        pltpu.sync_copy(hs_hbm.at[pl.ds(base, TPW)], rows_v)
        pltpu.sync_copy(p0_hbm.at[pl.ds(base, TPW)], idx_v)
        pltpu.async_copy(rows_v, xpad_hbm.at[idx_v], sem).wait()
        pltpu.sync_copy(p1_hbm.at[pl.ds(base, TPW)], idx_v)
        pltpu.async_copy(rows_v, xpad_hbm.at[idx_v], sem).wait()

    return k(hs2, pos0, pos1)


BH = S // NT


def _grouped_body(meta_ref, x_ref, w1_ref, w2_ref, hs_ref, gw_hbm, uw_hbm,
                  dw_hbm, y_ref, sh_ref, gw_v, uw_v, dw_v, wsem):
    j = pl.program_id(0)

    @pl.when(j == 0)
    def _():
        pltpu.make_async_copy(gw_hbm, gw_v, wsem).start()
        pltpu.make_async_copy(gw_hbm, gw_v, wsem).wait()
        pltpu.make_async_copy(uw_hbm, uw_v, wsem).start()
        pltpu.make_async_copy(uw_hbm, uw_v, wsem).wait()
        pltpu.make_async_copy(dw_hbm, dw_v, wsem).start()
        pltpu.make_async_copy(dw_hbm, dw_v, wsem).wait()

    xs = hs_ref[...]
    g = lax.dot_general(xs, gw_v[...], (((1,), (1,)), ((), ())),
                        preferred_element_type=jnp.float32)
    u = lax.dot_general(xs, uw_v[...], (((1,), (1,)), ((), ())),
                        preferred_element_type=jnp.float32)
    act_s = g * _sigmoid(g) * u
    sh_ref[...] = lax.dot_general(act_s, dw_v[...], (((1,), (1,)), ((), ())),
                                  preferred_element_type=jnp.float32)

    @pl.when(meta_ref[1, j] == 1)
    def _():
        x = x_ref[...]
        h = lax.dot_general(x, w1_ref[0], (((1,), (0,)), ((), ())),
                            preferred_element_type=jnp.float32)
        proj = h[:, :INTER]
        gate = h[:, INTER:]
        act = proj * _sigmoid(proj) * gate
        y_ref[...] = lax.dot_general(act, w2_ref[0], (((1,), (0,)), ((), ())),
                                     preferred_element_type=jnp.float32)


def _grouped(xpad, fc1_w, fc2_w, hs2, gate_w, up_w, down_w, meta):
    grid_spec = pltpu.PrefetchScalarGridSpec(
        num_scalar_prefetch=1,
        grid=(NT,),
        in_specs=[
            pl.BlockSpec((BT, HIDDEN), lambda j, m: (m[1, j] * j, 0)),
            pl.BlockSpec((1, HIDDEN, 2 * INTER), lambda j, m: (m[0, j], 0, 0)),
            pl.BlockSpec((1, INTER, HIDDEN), lambda j, m: (m[0, j], 0, 0)),
            pl.BlockSpec((BH, HIDDEN), lambda j, m: (j, 0)),
            pl.BlockSpec(memory_space=pl.ANY),
            pl.BlockSpec(memory_space=pl.ANY),
            pl.BlockSpec(memory_space=pl.ANY),
        ],
        out_specs=[
            pl.BlockSpec((BT, HIDDEN), lambda j, m: (j, 0)),
            pl.BlockSpec((BH, HIDDEN), lambda j, m: (j, 0)),
        ],
        scratch_shapes=[
            pltpu.VMEM((2 * INTER, HIDDEN), jnp.float32),
            pltpu.VMEM((2 * INTER, HIDDEN), jnp.float32),
            pltpu.VMEM((HIDDEN, 2 * INTER), jnp.float32),
            pltpu.SemaphoreType.DMA,
        ],
    )
    return pl.pallas_call(
        _grouped_body,
        grid_spec=grid_spec,
        out_shape=[
            jax.ShapeDtypeStruct((NPAD, HIDDEN), jnp.float32),
            jax.ShapeDtypeStruct((S, HIDDEN), jnp.float32),
        ],
    )(meta, xpad, fc1_w, fc2_w, hs2, gate_w, up_w, down_w)


def _sc_gather(ypad, pos0, pos1):
    mesh = plsc.VectorSubcoreMesh(core_axis_name="c", subcore_axis_name="s")

    @functools.partial(
        pl.kernel, mesh=mesh,
        out_type=(
            jax.ShapeDtypeStruct((S, HIDDEN), jnp.float32),
            jax.ShapeDtypeStruct((S, HIDDEN), jnp.float32),
        ),
        scratch_types=[
            pltpu.VMEM((TPW,), jnp.int32),
            pltpu.VMEM((TPW, HIDDEN), jnp.float32),
            pltpu.SemaphoreType.DMA,
        ],
    )
    def k(ypad_hbm, p0_hbm, p1_hbm, g0_hbm, g1_hbm, idx_v, rows_v, sem):
        wid = lax.axis_index("s") * 2 + lax.axis_index("c")
        base = wid * TPW
        pltpu.sync_copy(p0_hbm.at[pl.ds(base, TPW)], idx_v)
        pltpu.async_copy(ypad_hbm.at[idx_v], rows_v, sem).wait()
        pltpu.sync_copy(rows_v, g0_hbm.at[pl.ds(base, TPW)])
        pltpu.sync_copy(p1_hbm.at[pl.ds(base, TPW)], idx_v)
        pltpu.async_copy(ypad_hbm.at[idx_v], rows_v, sem).wait()
        pltpu.sync_copy(rows_v, g1_hbm.at[pl.ds(base, TPW)])

    return k(ypad, pos0, pos1)


def _combine_body(sh_ref, g0_ref, g1_ref, sc_ref, o_ref):
    s = sc_ref[...]
    o_ref[...] = (sh_ref[...] + s[0][:, None] * g0_ref[...]
                  + s[1][:, None] * g1_ref[...])


def _combine(sh, g0, g1, scores):
    BS = 512
    nb = S // BS
    return pl.pallas_call(
        _combine_body,
        grid=(nb,),
        in_specs=[
            pl.BlockSpec((BS, HIDDEN), lambda j: (j, 0)),
            pl.BlockSpec((BS, HIDDEN), lambda j: (j, 0)),
            pl.BlockSpec((BS, HIDDEN), lambda j: (j, 0)),
            pl.BlockSpec((2, BS), lambda j: (0, j)),
        ],
        out_specs=pl.BlockSpec((BS, HIDDEN), lambda j: (j, 0)),
        out_shape=jax.ShapeDtypeStruct((S, HIDDEN), jnp.float32),
    )(sh, g0, g1, scores)


def kernel(hidden_states, router_w, fc1_w, fc2_w, gate_w, up_w, down_w):
    orig_shape = hidden_states.shape
    hs2 = hidden_states.reshape(S, HIDDEN)
    scores, pos, meta = _router(hs2, router_w)
    pos0 = pos[0]
    pos1 = pos[1]
    xpad = _sc_dispatch(hs2, pos0, pos1)
    ypad, sh = _grouped(xpad, fc1_w, fc2_w, hs2, gate_w, up_w, down_w, meta)
    g0, g1 = _sc_gather(ypad, pos0, pos1)
    out = _combine(sh, g0, g1, scores)
    return out.reshape(orig_shape)


# speedup vs baseline: 1.5184x; 1.5184x over previous
import functools

import jax
import jax.numpy as jnp
from jax import lax
from jax.experimental import pallas as pl
from jax.experimental.pallas import tpu as pltpu
from jax.experimental.pallas import tpu_sc as plsc

HIDDEN = 768
INTER = 1024
NUM_EXPERTS = 64
TOPK = 2
S = 2048
SLOTS = S * TOPK
BT = 64
NT = NUM_EXPERTS + SLOTS // BT
NPAD = NT * BT
NW = 32
TPW = S // NW


def _sigmoid(x):
    return 1.0 / (1.0 + jnp.exp(-x))


def _router_body(hs_ref, rw_ref, scores_ref, pos_ref, meta_ref):
    x = hs_ref[...]
    w = rw_ref[...]
    logits = lax.dot_general(x, w, (((1,), (1,)), ((), ())),
                             preferred_element_type=jnp.float32)
    iota_e = lax.broadcasted_iota(jnp.int32, (S, NUM_EXPERTS), 1)
    big = jnp.int32(2**30)
    m0 = jnp.max(logits, axis=1)
    a0 = jnp.min(jnp.where(logits == m0[:, None], iota_e, big), axis=1)
    masked = jnp.where(iota_e == a0[:, None], -jnp.inf, logits)
    m1 = jnp.max(masked, axis=1)
    a1 = jnp.min(jnp.where(masked == m1[:, None], iota_e, big), axis=1)
    e1 = jnp.exp(m1 - m0)
    scores_ref[0, :] = 1.0 / (1.0 + e1)
    scores_ref[1, :] = e1 / (1.0 + e1)

    oh0 = (a0[:, None] == iota_e).astype(jnp.float32)
    oh1 = (a1[:, None] == iota_e).astype(jnp.float32)

    CB = 256
    r_i = lax.broadcasted_iota(jnp.int32, (CB, CB), 0)
    c_i = lax.broadcasted_iota(jnp.int32, (CB, CB), 1)
    l_incl = (r_i >= c_i).astype(jnp.float32)
    carry = jnp.zeros((1, NUM_EXPERTS), jnp.float32)
    ranks = []
    for half in (oh0, oh1):
        for k in range(S // CB):
            blk = half[k * CB:(k + 1) * CB, :]
            cum_in = lax.dot_general(l_incl, blk, (((1,), (0,)), ((), ())),
                                     preferred_element_type=jnp.float32) + carry
            rank_rows = cum_in - blk
            ranks.append(jnp.sum(rank_rows * blk, axis=1))
            carry = carry + jnp.sum(blk, axis=0, keepdims=True)
    counts = carry
    nchunk = S // CB

    n_e = jnp.floor((counts + (BT - 1)) * (1.0 / BT))
    iu_r = lax.broadcasted_iota(jnp.int32, (NUM_EXPERTS, NUM_EXPERTS), 0)
    iu_c = lax.broadcasted_iota(jnp.int32, (NUM_EXPERTS, NUM_EXPERTS), 1)
    u_incl = (iu_r <= iu_c).astype(jnp.float32)
    cumn_incl = lax.dot_general(n_e, u_incl, (((1,), (0,)), ((), ())),
                                preferred_element_type=jnp.float32)
    cumn_excl = cumn_incl - n_e
    base_e = cumn_excl * float(BT)

    base0 = jnp.sum(oh0 * base_e, axis=1)
    base1 = jnp.sum(oh1 * base_e, axis=1)
    for k in range(nchunk):
        sl = slice(k * CB, (k + 1) * CB)
        pos_ref[0, sl] = (base0[sl] + ranks[k]).astype(jnp.int32)
        pos_ref[1, sl] = (base1[sl] + ranks[nchunk + k]).astype(jnp.int32)

    jt = lax.broadcasted_iota(jnp.int32, (NT, NUM_EXPERTS), 0).astype(jnp.float32)
    owner = jnp.sum((jnp.broadcast_to(cumn_incl, (NT, NUM_EXPERTS)) <= jt)
                    .astype(jnp.int32), axis=1)
    valid = (owner < NUM_EXPERTS).astype(jnp.int32)
    iota_e1 = lax.broadcasted_iota(jnp.int32, (1, NUM_EXPERTS), 1)
    last_e = jnp.max(jnp.where(counts > 0.0, iota_e1, 0))
    meta_ref[0, :] = jnp.where(valid == 1, owner, last_e)
    meta_ref[1, :] = valid


def _router(hs2, router_w):
    return pl.pallas_call(
        _router_body,
        out_shape=(
            jax.ShapeDtypeStruct((2, S), jnp.float32),
            jax.ShapeDtypeStruct((2, S), jnp.int32),
            jax.ShapeDtypeStruct((2, NT), jnp.int32),
        ),
    )(hs2, router_w)


def _sc_dispatch(hs2, pos0, pos1):
    mesh = plsc.VectorSubcoreMesh(core_axis_name="c", subcore_axis_name="s")

    @functools.partial(
        pl.kernel, mesh=mesh,
        out_type=jax.ShapeDtypeStruct((NPAD, HIDDEN), jnp.float32),
        scratch_types=[
            pltpu.VMEM((TPW,), jnp.int32),
            pltpu.VMEM((TPW, HIDDEN), jnp.float32),
            pltpu.SemaphoreType.DMA,
        ],
    )
    def k(hs_hbm, p0_hbm, p1_hbm, xpad_hbm, idx_v, rows_v, sem):
        wid = lax.axis_index("s") * 2 + lax.axis_index("c")
        base = wid * TPW
        pltpu.sync_copy(hs_hbm.at[pl.ds(base, TPW)], rows_v)
        pltpu.sync_copy(p0_hbm.at[pl.ds(base, TPW)], idx_v)
        pltpu.async_copy(rows_v, xpad_hbm.at[idx_v], sem).wait()
        pltpu.sync_copy(p1_hbm.at[pl.ds(base, TPW)], idx_v)
        pltpu.async_copy(rows_v, xpad_hbm.at[idx_v], sem).wait()

    return k(hs2, pos0, pos1)


BH = 256
SHARED_EVERY = NT // (S // BH)
SHARED_AT = 1


def _grouped_body(meta_ref, x_ref, w1_ref, w2_ref, hs_ref, gw_hbm, uw_hbm,
                  dw_hbm, y_ref, sh_ref, gw_v, uw_v, dw_v, wsem):
    j = pl.program_id(0)

    @pl.when(j == 0)
    def _():
        pltpu.make_async_copy(gw_hbm, gw_v, wsem).start()
        pltpu.make_async_copy(uw_hbm, uw_v, wsem).start()
        pltpu.make_async_copy(dw_hbm, dw_v, wsem).start()

    @pl.when(j == SHARED_AT)
    def _():
        pltpu.make_async_copy(gw_hbm, gw_v, wsem).wait()
        pltpu.make_async_copy(uw_hbm, uw_v, wsem).wait()
        pltpu.make_async_copy(dw_hbm, dw_v, wsem).wait()

    @pl.when(j % SHARED_EVERY == SHARED_AT)
    def _():
        xs = hs_ref[...]
        g = lax.dot_general(xs, gw_v[...], (((1,), (1,)), ((), ())),
                            preferred_element_type=jnp.float32)
        u = lax.dot_general(xs, uw_v[...], (((1,), (1,)), ((), ())),
                            preferred_element_type=jnp.float32)
        act_s = g * _sigmoid(g) * u
        sh_ref[...] = lax.dot_general(act_s, dw_v[...], (((1,), (1,)), ((), ())),
                                      preferred_element_type=jnp.float32)

    @pl.when(meta_ref[1, j] == 1)
    def _():
        x = x_ref[...]
        h = lax.dot_general(x, w1_ref[0], (((1,), (0,)), ((), ())),
                            preferred_element_type=jnp.float32)
        proj = h[:, :INTER]
        gate = h[:, INTER:]
        act = proj * _sigmoid(proj) * gate
        y_ref[...] = lax.dot_general(act, w2_ref[0], (((1,), (0,)), ((), ())),
                                     preferred_element_type=jnp.float32)


def _grouped(xpad, fc1_w, fc2_w, hs2, gate_w, up_w, down_w, meta):
    grid_spec = pltpu.PrefetchScalarGridSpec(
        num_scalar_prefetch=1,
        grid=(NT,),
        in_specs=[
            pl.BlockSpec((BT, HIDDEN), lambda j, m: (m[1, j] * j, 0)),
            pl.BlockSpec((1, HIDDEN, 2 * INTER), lambda j, m: (m[0, j], 0, 0)),
            pl.BlockSpec((1, INTER, HIDDEN), lambda j, m: (m[0, j], 0, 0)),
            pl.BlockSpec((BH, HIDDEN), lambda j, m: (j // SHARED_EVERY, 0)),
            pl.BlockSpec(memory_space=pl.ANY),
            pl.BlockSpec(memory_space=pl.ANY),
            pl.BlockSpec(memory_space=pl.ANY),
        ],
        out_specs=[
            pl.BlockSpec((BT, HIDDEN), lambda j, m: (j, 0)),
            pl.BlockSpec((BH, HIDDEN), lambda j, m: (j // SHARED_EVERY, 0)),
        ],
        scratch_shapes=[
            pltpu.VMEM((2 * INTER, HIDDEN), jnp.float32),
            pltpu.VMEM((2 * INTER, HIDDEN), jnp.float32),
            pltpu.VMEM((HIDDEN, 2 * INTER), jnp.float32),
            pltpu.SemaphoreType.DMA,
        ],
    )
    return pl.pallas_call(
        _grouped_body,
        grid_spec=grid_spec,
        out_shape=[
            jax.ShapeDtypeStruct((NPAD, HIDDEN), jnp.float32),
            jax.ShapeDtypeStruct((S, HIDDEN), jnp.float32),
        ],
    )(meta, xpad, fc1_w, fc2_w, hs2, gate_w, up_w, down_w)


def _sc_gather(ypad, pos0, pos1):
    mesh = plsc.VectorSubcoreMesh(core_axis_name="c", subcore_axis_name="s")

    @functools.partial(
        pl.kernel, mesh=mesh,
        out_type=(
            jax.ShapeDtypeStruct((S, HIDDEN), jnp.float32),
            jax.ShapeDtypeStruct((S, HIDDEN), jnp.float32),
        ),
        scratch_types=[
            pltpu.VMEM((TPW,), jnp.int32),
            pltpu.VMEM((TPW, HIDDEN), jnp.float32),
            pltpu.SemaphoreType.DMA,
        ],
    )
    def k(ypad_hbm, p0_hbm, p1_hbm, g0_hbm, g1_hbm, idx_v, rows_v, sem):
        wid = lax.axis_index("s") * 2 + lax.axis_index("c")
        base = wid * TPW
        pltpu.sync_copy(p0_hbm.at[pl.ds(base, TPW)], idx_v)
        pltpu.async_copy(ypad_hbm.at[idx_v], rows_v, sem).wait()
        pltpu.sync_copy(rows_v, g0_hbm.at[pl.ds(base, TPW)])
        pltpu.sync_copy(p1_hbm.at[pl.ds(base, TPW)], idx_v)
        pltpu.async_copy(ypad_hbm.at[idx_v], rows_v, sem).wait()
        pltpu.sync_copy(rows_v, g1_hbm.at[pl.ds(base, TPW)])

    return k(ypad, pos0, pos1)


def _combine_body(sh_ref, g0_ref, g1_ref, sc_ref, o_ref):
    s = sc_ref[...]
    o_ref[...] = (sh_ref[...] + s[0][:, None] * g0_ref[...]
                  + s[1][:, None] * g1_ref[...])


def _combine(sh, g0, g1, scores):
    BS = 512
    nb = S // BS
    return pl.pallas_call(
        _combine_body,
        grid=(nb,),
        in_specs=[
            pl.BlockSpec((BS, HIDDEN), lambda j: (j, 0)),
            pl.BlockSpec((BS, HIDDEN), lambda j: (j, 0)),
            pl.BlockSpec((BS, HIDDEN), lambda j: (j, 0)),
            pl.BlockSpec((2, BS), lambda j: (0, j)),
        ],
        out_specs=pl.BlockSpec((BS, HIDDEN), lambda j: (j, 0)),
        out_shape=jax.ShapeDtypeStruct((S, HIDDEN), jnp.float32),
    )(sh, g0, g1, scores)


def kernel(hidden_states, router_w, fc1_w, fc2_w, gate_w, up_w, down_w):
    orig_shape = hidden_states.shape
    hs2 = hidden_states.reshape(S, HIDDEN)
    scores, pos, meta = _router(hs2, router_w)
    pos0 = pos[0]
    pos1 = pos[1]
    xpad = _sc_dispatch(hs2, pos0, pos1)
    ypad, sh = _grouped(xpad, fc1_w, fc2_w, hs2, gate_w, up_w, down_w, meta)
    g0, g1 = _sc_gather(ypad, pos0, pos1)
    out = _combine(sh, g0, g1, scores)
    return out.reshape(orig_shape)
